# Initial kernel scaffold; baseline (speedup 1.0000x reference)
import jax
import jax.numpy as jnp
from jax.experimental import pallas as pl


def kernel(x, w_gate, w_noise, down_w, down_b, up_w, up_b):
    raise NotImplementedError("write your pallas kernel here")


# speedup vs baseline: 2.4352x; 2.4352x over previous
import functools

import jax
import jax.numpy as jnp
from jax.experimental import pallas as pl

E = 16
TOPK = 2
SCALE = 0.5
NEG = -1e30


def _gates_dense(logits):
    lane = jax.lax.broadcasted_iota(jnp.int32, logits.shape, 1)
    m1 = jnp.max(logits, axis=1, keepdims=True)
    i1 = jnp.min(jnp.where(logits == m1, lane, E), axis=1, keepdims=True)
    sel1 = lane == i1
    l2 = jnp.where(sel1, NEG, logits)
    m2 = jnp.max(l2, axis=1, keepdims=True)
    i2 = jnp.min(jnp.where(l2 == m2, lane, E), axis=1, keepdims=True)
    sel2 = lane == i2
    e21 = jnp.exp(m2 - m1)
    g1 = 1.0 / (1.0 + e21)
    g2 = 1.0 - g1
    return jnp.where(sel1, g1, 0.0) + jnp.where(sel2, g2, 0.0)


def _dense_body(x_ref, wg_ref, dw_ref, db_ref, uw_ref, ub_ref, o_ref):
    xb = x_ref[...]
    logits = jnp.dot(xb, wg_ref[...], preferred_element_type=jnp.float32)
    gates = _gates_dense(logits)
    blk, d = xb.shape
    acc = jnp.zeros((blk, d), jnp.float32)
    for e in range(E):
        h = jnp.dot(xb, dw_ref[e], preferred_element_type=jnp.float32)
        h = jnp.maximum(h + db_ref[e][None, :], 0.0)
        y = jnp.dot(h, uw_ref[e], preferred_element_type=jnp.float32)
        y = y + ub_ref[e][None, :]
        acc = acc + gates[:, e][:, None] * y
    o_ref[...] = acc * SCALE


@functools.partial(jax.jit, static_argnames=("interpret",))
def kernel(x, w_gate, w_noise, down_w, down_b, up_w, up_b, interpret=False):
    del w_noise
    t, d = x.shape
    blk = 256
    b = down_w.shape[-1]
    full = lambda shape: pl.BlockSpec(shape, lambda i: tuple(0 for _ in shape))
    return pl.pallas_call(
        _dense_body,
        grid=(t // blk,),
        in_specs=[
            pl.BlockSpec((blk, d), lambda i: (i, 0)),
            full((d, E)),
            full((E, d, b)),
            full((E, b)),
            full((E, b, d)),
            full((E, d)),
        ],
        out_specs=pl.BlockSpec((blk, d), lambda i: (i, 0)),
        out_shape=jax.ShapeDtypeStruct((t, d), jnp.float32),
        interpret=interpret,
    )(x, w_gate, down_w, down_b, up_w, up_b)
